# baseline (speedup 1.0000x reference)
import functools

import jax
import jax.numpy as jnp
from jax import lax
from jax.experimental import pallas as pl
from jax.experimental.pallas import tpu as pltpu
from jax.experimental.pallas import tpu_sc as plsc

B = 1024
M = 1024
EMB = 128
D_OUT = 2048
TAU = 0.07
GAMMA = 0.5

NC = 2
NS = 16
NW = NC * NS
B_PER_W = B // NW
CH = 128
CHUNKS = M // CH
STEPS = B_PER_W * CHUNKS


def _emb_body(x_ref, w_ref, b_ref, o_ref):
    e = jnp.dot(x_ref[...], w_ref[...], preferred_element_type=jnp.float32)
    e = e + b_ref[...]
    n = jnp.sqrt(jnp.sum(e * e, axis=1, keepdims=True))
    o_ref[...] = e / jnp.maximum(n, 1e-12)


def _emb_call(outputs, W, b2d):
    grid = 4
    rows = B // grid
    return pl.pallas_call(
        _emb_body,
        grid=(grid,),
        in_specs=[
            pl.BlockSpec((rows, D_OUT), lambda i: (i, 0)),
            pl.BlockSpec((D_OUT, EMB), lambda i: (0, 0)),
            pl.BlockSpec((1, EMB), lambda i: (0, 0)),
        ],
        out_specs=pl.BlockSpec((rows, EMB), lambda i: (i, 0)),
        out_shape=jax.ShapeDtypeStruct((B, EMB), jnp.float32),
    )(outputs, W, b2d)


NBUF = 2


def _sc_body(bank, emb, posidx, negidx, neg_out, pos_out,
             idxb, rowb, embv, outv, parts, pidxv, prowv,
             isems, rsems, osem, psem):
    wid = lax.axis_index("s") * NC + lax.axis_index("c")
    base = wid * STEPS
    bbase = wid * B_PER_W

    pltpu.sync_copy(emb.at[pl.ds(bbase, B_PER_W)], embv)

    for t in range(NBUF):
        pltpu.make_async_copy(negidx.at[base + t], idxb[t], isems[t]).start()
    for t in range(NBUF - 1):
        pltpu.make_async_copy(negidx.at[base + t], idxb[t], isems[t]).wait()
        pltpu.make_async_copy(bank.at[idxb[t]], rowb[t], rsems[t]).start()

    pltpu.sync_copy(posidx.at[pl.ds(bbase, B_PER_W)], pidxv)
    pltpu.make_async_copy(bank.at[pidxv], prowv, psem).start()

    lane = lax.iota(jnp.int32, 16)
    lane17 = lane * 17

    def step(s, r):
        rn = (r + NBUF - 1) % NBUF

        @pl.when(s + NBUF - 1 < STEPS)
        def _():
            pltpu.make_async_copy(negidx.at[base + s + NBUF - 1], idxb[rn],
                                  isems[rn]).wait()
            pltpu.make_async_copy(bank.at[idxb[rn]], rowb[rn],
                                  rsems[rn]).start()

        pltpu.make_async_copy(bank.at[idxb[r]], rowb[r], rsems[r]).wait()

        @pl.when(s + NBUF < STEPS)
        def _():
            pltpu.make_async_copy(negidx.at[base + s + NBUF], idxb[r],
                                  isems[r]).start()

        bl = s // CHUNKS
        c = s % CHUNKS

        @pl.when(jnp.logical_and(c == 0, bl > 0))
        def _():
            pltpu.make_async_copy(outv, neg_out.at[bbase + bl - 1],
                                  osem).wait()

        emb_vecs = [embv[bl, pl.ds(16 * k, 16)] for k in range(8)]
        row_cur = rowb[r]

        @plsc.parallel_loop(0, CH, unroll=8)
        def _(m):
            p = [row_cur[m, pl.ds(16 * k, 16)] * emb_vecs[k]
                 for k in range(8)]
            parts[pl.ds(m * 17, 16)] = \
                ((p[0] + p[1]) + (p[2] + p[3])) + \
                ((p[4] + p[5]) + (p[6] + p[7]))

        @plsc.parallel_loop(0, CH // 16, unroll=2)
        def _(g):
            lidx = lane17 + g * (16 * 17)
            cols = [plsc.load_gather(parts, [lidx + l])
                    for l in range(16)]
            t0 = ((cols[0] + cols[1]) + (cols[2] + cols[3])) + \
                 ((cols[4] + cols[5]) + (cols[6] + cols[7]))
            t1 = ((cols[8] + cols[9]) + (cols[10] + cols[11])) + \
                 ((cols[12] + cols[13]) + (cols[14] + cols[15]))
            outv[pl.ds(c * CH + g * 16, 16)] = t0 + t1

        @pl.when(c == CHUNKS - 1)
        def _():
            pltpu.make_async_copy(outv, neg_out.at[bbase + bl], osem).start()

    def body(q, carry):
        for r in range(NBUF):
            step(NBUF * q + r, r)
        return carry

    lax.fori_loop(0, STEPS // NBUF, body, 0)
    pltpu.make_async_copy(outv, neg_out.at[bbase + B_PER_W - 1], osem).wait()
    pltpu.make_async_copy(bank.at[pidxv], prowv, psem).wait()
    pltpu.sync_copy(prowv, pos_out.at[pl.ds(bbase, B_PER_W)])


_sc_call = functools.partial(
    pl.kernel,
    out_type=[
        jax.ShapeDtypeStruct((B, M), jnp.float32),
        jax.ShapeDtypeStruct((B, EMB), jnp.float32),
    ],
    mesh=plsc.VectorSubcoreMesh(core_axis_name="c", subcore_axis_name="s",
                                num_cores=NC, num_subcores=NS),
    compiler_params=pltpu.CompilerParams(needs_layout_passes=False),
    scratch_types=[
        [pltpu.VMEM((CH,), jnp.int32) for _ in range(NBUF)],
        [pltpu.VMEM((CH, EMB), jnp.float32) for _ in range(NBUF)],
        pltpu.VMEM((B_PER_W, EMB), jnp.float32),
        pltpu.VMEM((M,), jnp.float32),
        pltpu.VMEM((CH * 17,), jnp.float32),
        pltpu.VMEM((B_PER_W,), jnp.int32),
        pltpu.VMEM((B_PER_W, EMB), jnp.float32),
        [pltpu.SemaphoreType.DMA for _ in range(NBUF)],
        [pltpu.SemaphoreType.DMA for _ in range(NBUF)],
        pltpu.SemaphoreType.DMA,
        pltpu.SemaphoreType.DMA,
    ],
)(_sc_body)


_LG = 8


def _loss_body(neg_ref, pm_ref, emb_ref, loss_ref, upd_ref, dl_ref, nl_ref):
    i = pl.program_id(0)
    emb = emb_ref[...]
    pm = pm_ref[...]
    u_pos = jnp.sum(emb * pm, axis=1) / TAU
    u_neg = neg_ref[...] * (1.0 / TAU)

    mx = jnp.max(u_neg, axis=1)
    log_C = mx + jnp.log(jnp.sum(jnp.exp(u_neg - mx[:, None]), axis=1))

    mxd = jnp.maximum(u_pos, log_C)
    ldd = mxd + jnp.log(jnp.exp(u_pos - mxd) + jnp.exp(log_C - mxd))
    data_part = -jnp.sum(u_pos - ldd) / B

    d = u_neg - log_C[:, None]
    noise_part = jnp.sum(jnp.maximum(d, 0.0) +
                         jnp.log(1.0 + jnp.exp(-jnp.abs(d)))) / B

    @pl.when(i == 0)
    def _():
        loss_ref[...] = jnp.zeros((1, 1), jnp.float32)
        dl_ref[...] = jnp.zeros((1, 1), jnp.float32)
        nl_ref[...] = jnp.zeros((1, 1), jnp.float32)

    loss_ref[...] += jnp.reshape(data_part + noise_part, (1, 1))
    dl_ref[...] += jnp.reshape(data_part, (1, 1))
    nl_ref[...] += jnp.reshape(noise_part, (1, 1))

    upd = GAMMA * pm + (1.0 - GAMMA) * emb
    n = jnp.sqrt(jnp.sum(upd * upd, axis=1, keepdims=True))
    upd_ref[...] = upd / jnp.maximum(n, 1e-12)


def _loss_call(neg_inner, pos_mem, emb):
    rows = B // _LG
    return pl.pallas_call(
        _loss_body,
        grid=(_LG,),
        in_specs=[
            pl.BlockSpec((rows, M), lambda i: (i, 0)),
            pl.BlockSpec((rows, EMB), lambda i: (i, 0)),
            pl.BlockSpec((rows, EMB), lambda i: (i, 0)),
        ],
        out_specs=[
            pl.BlockSpec((1, 1), lambda i: (0, 0)),
            pl.BlockSpec((rows, EMB), lambda i: (i, 0)),
            pl.BlockSpec((1, 1), lambda i: (0, 0)),
            pl.BlockSpec((1, 1), lambda i: (0, 0)),
        ],
        out_shape=[
            jax.ShapeDtypeStruct((1, 1), jnp.float32),
            jax.ShapeDtypeStruct((B, EMB), jnp.float32),
            jax.ShapeDtypeStruct((1, 1), jnp.float32),
            jax.ShapeDtypeStruct((1, 1), jnp.float32),
        ],
    )(neg_inner, pos_mem, emb)


def kernel(outputs, indices, memory_bank, W, b, neg_idxs):
    emb = _emb_call(outputs.astype(jnp.float32), W, b.reshape(1, EMB))
    neg_flat = neg_idxs.astype(jnp.int32).reshape(B * CHUNKS, CH)
    neg_inner, pos_mem = _sc_call(memory_bank, emb,
                                  indices.astype(jnp.int32), neg_flat)
    loss, upd, dl, nl = _loss_call(neg_inner, pos_mem, emb)
    return loss[0, 0], upd, dl[0, 0], nl[0, 0]


# speedup vs baseline: 1.0157x; 1.0157x over previous
import functools

import jax
import jax.numpy as jnp
from jax import lax
from jax.experimental import pallas as pl
from jax.experimental.pallas import tpu as pltpu
from jax.experimental.pallas import tpu_sc as plsc

B = 1024
M = 1024
EMB = 128
D_OUT = 2048
TAU = 0.07
GAMMA = 0.5

NC = 2
NS = 16
NW = NC * NS
B_PER_W = B // NW
CH = 128
CHUNKS = M // CH
STEPS = B_PER_W * CHUNKS


def _emb_body(x_ref, w_ref, b_ref, o_ref):
    e = jnp.dot(x_ref[...], w_ref[...], preferred_element_type=jnp.float32)
    e = e + b_ref[...]
    n = jnp.sqrt(jnp.sum(e * e, axis=1, keepdims=True))
    o_ref[...] = e / jnp.maximum(n, 1e-12)


def _emb_call(outputs, W, b2d):
    grid = 4
    rows = B // grid
    return pl.pallas_call(
        _emb_body,
        grid=(grid,),
        in_specs=[
            pl.BlockSpec((rows, D_OUT), lambda i: (i, 0)),
            pl.BlockSpec((D_OUT, EMB), lambda i: (0, 0)),
            pl.BlockSpec((1, EMB), lambda i: (0, 0)),
        ],
        out_specs=pl.BlockSpec((rows, EMB), lambda i: (i, 0)),
        out_shape=jax.ShapeDtypeStruct((B, EMB), jnp.float32),
    )(outputs, W, b2d)


NBUF = 2


def _sc_body(bank, emb, posidx, negidx, neg_out, pos_out,
             idxb, rowb, embv, outv, parts, pidxv, prowv,
             isems, rsems, osem, psem):
    wid = lax.axis_index("s") * NC + lax.axis_index("c")
    base = wid * STEPS
    bbase = wid * B_PER_W

    pltpu.sync_copy(emb.at[pl.ds(bbase, B_PER_W)], embv)

    for t in range(NBUF):
        pltpu.make_async_copy(negidx.at[base + t], idxb[t], isems[t]).start()
    for t in range(NBUF - 1):
        pltpu.make_async_copy(negidx.at[base + t], idxb[t], isems[t]).wait()
        pltpu.make_async_copy(bank.at[idxb[t]], rowb[t], rsems[t]).start()

    pltpu.sync_copy(posidx.at[pl.ds(bbase, B_PER_W)], pidxv)
    pltpu.make_async_copy(bank.at[pidxv], prowv, psem).start()

    lane = lax.iota(jnp.int32, 16)
    lane17 = lane * 17

    def step(s, r):
        rn = (r + NBUF - 1) % NBUF

        @pl.when(s + NBUF - 1 < STEPS)
        def _():
            pltpu.make_async_copy(negidx.at[base + s + NBUF - 1], idxb[rn],
                                  isems[rn]).wait()
            pltpu.make_async_copy(bank.at[idxb[rn]], rowb[rn],
                                  rsems[rn]).start()

        pltpu.make_async_copy(bank.at[idxb[r]], rowb[r], rsems[r]).wait()

        @pl.when(s + NBUF < STEPS)
        def _():
            pltpu.make_async_copy(negidx.at[base + s + NBUF], idxb[r],
                                  isems[r]).start()

        bl = s // CHUNKS
        c = s % CHUNKS

        @pl.when(jnp.logical_and(c == 0, bl > 0))
        def _():
            pltpu.make_async_copy(outv, neg_out.at[bbase + bl - 1],
                                  osem).wait()

        emb_vecs = [embv[bl, pl.ds(16 * k, 16)] for k in range(8)]
        row_cur = rowb[r]

        @plsc.parallel_loop(0, CH, unroll=8)
        def _(m):
            p = [row_cur[m, pl.ds(16 * k, 16)] * emb_vecs[k]
                 for k in range(8)]
            parts[pl.ds(m * 17, 16)] = \
                ((p[0] + p[1]) + (p[2] + p[3])) + \
                ((p[4] + p[5]) + (p[6] + p[7]))

        @plsc.parallel_loop(0, CH // 16, unroll=2)
        def _(g):
            lidx = lane17 + g * (16 * 17)
            cols = [plsc.load_gather(parts, [lidx + l])
                    for l in range(16)]
            t0 = ((cols[0] + cols[1]) + (cols[2] + cols[3])) + \
                 ((cols[4] + cols[5]) + (cols[6] + cols[7]))
            t1 = ((cols[8] + cols[9]) + (cols[10] + cols[11])) + \
                 ((cols[12] + cols[13]) + (cols[14] + cols[15]))
            outv[pl.ds(c * CH + g * 16, 16)] = t0 + t1

        @pl.when(c == CHUNKS - 1)
        def _():
            pltpu.make_async_copy(outv, neg_out.at[bbase + bl], osem).start()

    def body(q, carry):
        for r in range(NBUF):
            step(NBUF * q + r, r)
        return carry

    lax.fori_loop(0, STEPS // NBUF, body, 0)
    pltpu.make_async_copy(outv, neg_out.at[bbase + B_PER_W - 1], osem).wait()
    pltpu.make_async_copy(bank.at[pidxv], prowv, psem).wait()
    pltpu.sync_copy(prowv, pos_out.at[pl.ds(bbase, B_PER_W)])


_sc_call = functools.partial(
    pl.kernel,
    out_type=[
        jax.ShapeDtypeStruct((B, M), jnp.float32),
        jax.ShapeDtypeStruct((B, EMB), jnp.float32),
    ],
    mesh=plsc.VectorSubcoreMesh(core_axis_name="c", subcore_axis_name="s",
                                num_cores=NC, num_subcores=NS),
    compiler_params=pltpu.CompilerParams(needs_layout_passes=False),
    scratch_types=[
        [pltpu.VMEM((CH,), jnp.int32) for _ in range(NBUF)],
        [pltpu.VMEM((CH, EMB), jnp.float32) for _ in range(NBUF)],
        pltpu.VMEM((B_PER_W, EMB), jnp.float32),
        pltpu.VMEM((M,), jnp.float32),
        pltpu.VMEM((CH * 17,), jnp.float32),
        pltpu.VMEM((B_PER_W,), jnp.int32),
        pltpu.VMEM((B_PER_W, EMB), jnp.float32),
        [pltpu.SemaphoreType.DMA for _ in range(NBUF)],
        [pltpu.SemaphoreType.DMA for _ in range(NBUF)],
        pltpu.SemaphoreType.DMA,
        pltpu.SemaphoreType.DMA,
    ],
)(_sc_body)


def _loss_body(neg_ref, pm_ref, emb_ref, loss_ref, upd_ref, dl_ref, nl_ref):
    emb = emb_ref[...]
    pm = pm_ref[...]
    u_pos = jnp.sum(emb * pm, axis=1) / TAU
    u_neg = neg_ref[...] * (1.0 / TAU)

    mx = jnp.max(u_neg, axis=1)
    log_C = mx + jnp.log(jnp.sum(jnp.exp(u_neg - mx[:, None]), axis=1))

    mxd = jnp.maximum(u_pos, log_C)
    ldd = mxd + jnp.log(jnp.exp(u_pos - mxd) + jnp.exp(log_C - mxd))
    data_loss = -jnp.sum(u_pos - ldd) / B

    d = u_neg - log_C[:, None]
    noise_loss = jnp.sum(jnp.maximum(d, 0.0) +
                         jnp.log(1.0 + jnp.exp(-jnp.abs(d)))) / B

    loss_ref[...] = jnp.reshape(data_loss + noise_loss, (1, 1))
    dl_ref[...] = jnp.reshape(data_loss, (1, 1))
    nl_ref[...] = jnp.reshape(noise_loss, (1, 1))

    upd = GAMMA * pm + (1.0 - GAMMA) * emb
    n = jnp.sqrt(jnp.sum(upd * upd, axis=1, keepdims=True))
    upd_ref[...] = upd / jnp.maximum(n, 1e-12)


def _loss_call(neg_inner, pos_mem, emb):
    return pl.pallas_call(
        _loss_body,
        out_shape=[
            jax.ShapeDtypeStruct((1, 1), jnp.float32),
            jax.ShapeDtypeStruct((B, EMB), jnp.float32),
            jax.ShapeDtypeStruct((1, 1), jnp.float32),
            jax.ShapeDtypeStruct((1, 1), jnp.float32),
        ],
    )(neg_inner, pos_mem, emb)


def kernel(outputs, indices, memory_bank, W, b, neg_idxs):
    emb = _emb_call(outputs.astype(jnp.float32), W, b.reshape(1, EMB))
    neg_flat = neg_idxs.astype(jnp.int32).reshape(B * CHUNKS, CH)
    neg_inner, pos_mem = _sc_call(memory_bank, emb,
                                  indices.astype(jnp.int32), neg_flat)
    loss, upd, dl, nl = _loss_call(neg_inner, pos_mem, emb)
    return loss[0, 0], upd, dl[0, 0], nl[0, 0]
